# Initial kernel scaffold; baseline (speedup 1.0000x reference)
import jax
import jax.numpy as jnp
from jax.experimental import pallas as pl


def kernel(feat, graph_ids, num_graphs):
    raise NotImplementedError("write your pallas kernel here")


# speedup vs baseline: 4.5265x; 4.5265x over previous
import functools

import jax
import jax.numpy as jnp
from jax import lax
from jax.experimental import pallas as pl
from jax.experimental.pallas import tpu as pltpu
from jax.experimental.pallas import tpu_sc as plsc

N = 100000
D = 128
G = 256

NC = 2
NS = 16
L = 16

DH = D // NC
SPAN = 8 * -(-N // (NS * 8))
C = 512
SUB = 128
NCH = -(-SPAN // C)
GP = G + NS
STRIPE = GP // NS
GSEG = G // NS


def _body(feat_hbm, ids_hbm, out_hbm,
          feat_buf, ids_lin, ids2d, ones_buf, zbuf, czbuf,
          sum_buf, cnt_buf, out_buf, acc_sh, cnt_sh):
    c = lax.axis_index("c")
    s = lax.axis_index("s")
    col0 = c * DH

    ones16 = jnp.ones((L,), jnp.float32)
    zero16 = jnp.zeros((L,), jnp.float32)
    pos = lax.iota(jnp.int32, L)

    for r in range(SUB):
        ones_buf[r] = ones16
    for r in range(STRIPE):
        czbuf[r] = zero16
        for q in range(DH // L):
            zbuf[r, pl.ds(q * L, L)] = zero16

    pltpu.sync_copy(zbuf, acc_sh.at[pl.ds(s * STRIPE, STRIPE)])
    pltpu.sync_copy(czbuf, cnt_sh.at[pl.ds(s * STRIPE, STRIPE)])
    plsc.subcore_barrier()

    start = s * SPAN
    end = jnp.minimum(start + SPAN, N)

    def chunk(k, carry):
        lo_un = start + k * C
        lo = jnp.minimum(lo_un, end - C)
        delta = lo_un - lo
        pltpu.sync_copy(ids_hbm.at[pl.ds(lo, C)], ids_lin)
        pltpu.sync_copy(feat_hbm.at[pl.ds(lo, C), pl.ds(col0, DH)], feat_buf)
        for i in range(C // L):
            v = ids_lin[pl.ds(i * L, L)]
            keep = (pos + (i * L)) >= delta
            ids2d[i // (SUB // L), pl.ds((i % (SUB // L)) * L, L)] = (
                jnp.where(keep, v, G))
        for j in range(C // SUB):
            idxr = ids2d.at[j]
            pltpu.sync_copy(feat_buf.at[pl.ds(j * SUB, SUB)],
                            acc_sh.at[idxr], add=True)
            pltpu.sync_copy(ones_buf, cnt_sh.at[idxr], add=True)
        return carry

    lax.fori_loop(0, NCH, chunk, 0)
    plsc.subcore_barrier()

    g0 = s * GSEG
    pltpu.sync_copy(acc_sh.at[pl.ds(g0, GSEG)], sum_buf)
    pltpu.sync_copy(cnt_sh.at[pl.ds(g0, GSEG)], cnt_buf)
    for g in range(GSEG):
        recip = 1.0 / jnp.maximum(cnt_buf[g], 1.0)
        for q in range(DH // L):
            out_buf[g, pl.ds(q * L, L)] = sum_buf[g, pl.ds(q * L, L)] * recip
    pltpu.sync_copy(out_buf, out_hbm.at[pl.ds(g0, GSEG), pl.ds(col0, DH)])


@jax.jit
def _pooled(feat, graph_ids):
    mesh = plsc.VectorSubcoreMesh(core_axis_name="c", subcore_axis_name="s")
    f = pl.kernel(
        _body,
        out_type=jax.ShapeDtypeStruct((G, D), jnp.float32),
        mesh=mesh,
        compiler_params=pltpu.CompilerParams(use_tc_tiling_on_sc=False),
        scratch_types=[
            pltpu.VMEM((C, DH), jnp.float32),
            pltpu.VMEM((C,), jnp.int32),
            pltpu.VMEM((C // SUB, SUB), jnp.int32),
            pltpu.VMEM((SUB, L), jnp.float32),
            pltpu.VMEM((STRIPE, DH), jnp.float32),
            pltpu.VMEM((STRIPE, L), jnp.float32),
            pltpu.VMEM((GSEG, DH), jnp.float32),
            pltpu.VMEM((GSEG, L), jnp.float32),
            pltpu.VMEM((GSEG, DH), jnp.float32),
            pltpu.VMEM_SHARED((GP, DH), jnp.float32),
            pltpu.VMEM_SHARED((GP, L), jnp.float32),
        ],
    )
    return f(feat, graph_ids.astype(jnp.int32))


def kernel(feat, graph_ids, num_graphs):
    pooled = _pooled(feat, graph_ids)
    valid = jnp.arange(G)[:, None] < num_graphs
    return jnp.where(valid, pooled, jnp.zeros_like(pooled))
